# baseline (speedup 1.0000x reference)
import jax
import jax.numpy as jnp
from jax.experimental import pallas as pl
from jax.experimental.pallas import tpu as pltpu


_TOKENS_PER_STEP = 4096
_ROWS_PER_ITER = 16


def _round_up(x, m):
    return (x + m - 1) // m * m


def _make_gather_body(tokens_per_step, d2):
    h2 = 2 * d2

    def _body(ids_ref, tbl_ref, out_ref):
        tok0 = pl.program_id(0) * tokens_per_step
        lane = jax.lax.broadcasted_iota(jnp.int32, (1, h2), 1)
        left = lane < d2

        def chunk(c, carry):
            base = tok0 + c * (2 * _ROWS_PER_ITER)
            row0 = c * _ROWS_PER_ITER
            rows = []
            for u in range(_ROWS_PER_ITER):
                i0 = ids_ref[base + 2 * u]
                i1 = ids_ref[base + 2 * u + 1]
                a = pltpu.roll(tbl_ref[i0 >> 1], (i0 & 1) * d2, axis=1)
                bb = pltpu.roll(tbl_ref[i1 >> 1], (1 - (i1 & 1)) * d2, axis=1)
                rows.append(jnp.where(left, a, bb))
            for g in range(_ROWS_PER_ITER // 8):
                out_ref[pl.ds(pl.multiple_of(row0 + 8 * g, 8), 8), :] = (
                    jnp.concatenate(rows[8 * g:8 * g + 8], axis=0))
            return carry

        n_iters = tokens_per_step // (2 * _ROWS_PER_ITER)
        jax.lax.fori_loop(0, n_iters, chunk, 0)

    return _body


@jax.jit
def kernel(embedding, feature_embed, item_feature, inputs, A, combined):
    del feature_embed, item_feature, A
    n_node = embedding.shape[0]
    n_pad, d2 = combined.shape
    b, s = inputs.shape
    n = b * s
    h2 = 2 * d2

    ids = jnp.clip(inputs.reshape(-1).astype(jnp.int32), 0, n_node - 1)
    np_tok = _round_up(n, _TOKENS_PER_STEP)
    if np_tok != n:
        ids = jnp.pad(ids, (0, np_tok - n))
    grid = (np_tok // _TOKENS_PER_STEP,)
    rows_step = _TOKENS_PER_STEP // 2

    tbl = combined.reshape(n_pad // 2, 1, h2)

    out = pl.pallas_call(
        _make_gather_body(_TOKENS_PER_STEP, d2),
        out_shape=jax.ShapeDtypeStruct((np_tok // 2, h2), jnp.float32),
        grid=grid,
        in_specs=[
            pl.BlockSpec(memory_space=pltpu.SMEM),
            pl.BlockSpec((n_pad // 2, 1, h2), lambda i: (0, 0, 0)),
        ],
        out_specs=pl.BlockSpec((rows_step, h2), lambda i: (i, 0)),
        compiler_params=pltpu.CompilerParams(
            dimension_semantics=("parallel",),
            vmem_limit_bytes=int(56 << 20),
        ),
    )(ids, tbl)

    return out.reshape(np_tok, d2)[:n].reshape(b, s, d2)


# speedup vs baseline: 2.2888x; 2.2888x over previous
import jax
import jax.numpy as jnp
from jax.experimental import pallas as pl
from jax.experimental.pallas import tpu as pltpu


_TOKENS_PER_STEP = 8192
_ROWS_PER_ITER = 32


def _round_up(x, m):
    return (x + m - 1) // m * m


def _make_gather_body(tokens_per_step, d2):
    h2 = 2 * d2

    def _body(ids_ref, tbl_ref, out_ref):
        tok0 = pl.program_id(0) * tokens_per_step
        lane = jax.lax.broadcasted_iota(jnp.int32, (1, h2), 1)
        left = lane < d2

        def chunk(c, carry):
            base = tok0 + c * (2 * _ROWS_PER_ITER)
            row0 = c * _ROWS_PER_ITER
            rows = []
            for u in range(_ROWS_PER_ITER):
                i0 = ids_ref[base + 2 * u]
                i1 = ids_ref[base + 2 * u + 1]
                rows.append(jnp.where(left, tbl_ref[i0], tbl_ref[i1]))
            for g in range(_ROWS_PER_ITER // 8):
                out_ref[pl.ds(pl.multiple_of(row0 + 8 * g, 8), 8), :] = (
                    jnp.concatenate(rows[8 * g:8 * g + 8], axis=0))
            return carry

        n_iters = tokens_per_step // (2 * _ROWS_PER_ITER)
        jax.lax.fori_loop(0, n_iters, chunk, 0)

    return _body


@jax.jit
def kernel(embedding, feature_embed, item_feature, inputs, A, combined):
    del feature_embed, item_feature, A
    n_node = embedding.shape[0]
    n_pad, d2 = combined.shape
    b, s = inputs.shape
    n = b * s
    h2 = 2 * d2

    ids = jnp.clip(inputs.reshape(-1).astype(jnp.int32), 0, n_node - 1)
    np_tok = _round_up(n, _TOKENS_PER_STEP)
    if np_tok != n:
        ids = jnp.pad(ids, (0, np_tok - n))
    grid = (np_tok // _TOKENS_PER_STEP,)
    rows_step = _TOKENS_PER_STEP // 2

    tbl = jnp.concatenate([combined, combined], axis=1).reshape(n_pad, 1, h2)

    out = pl.pallas_call(
        _make_gather_body(_TOKENS_PER_STEP, d2),
        out_shape=jax.ShapeDtypeStruct((np_tok // 2, h2), jnp.float32),
        grid=grid,
        in_specs=[
            pl.BlockSpec(memory_space=pltpu.SMEM),
            pl.BlockSpec((n_pad, 1, h2), lambda i: (0, 0, 0)),
        ],
        out_specs=pl.BlockSpec((rows_step, h2), lambda i: (i, 0)),
        compiler_params=pltpu.CompilerParams(
            dimension_semantics=("parallel",),
            vmem_limit_bytes=int(56 << 20),
        ),
    )(ids, tbl)

    return out.reshape(np_tok, d2)[:n].reshape(b, s, d2)


# speedup vs baseline: 2.6682x; 1.1658x over previous
import jax
import jax.numpy as jnp
from jax.experimental import pallas as pl
from jax.experimental.pallas import tpu as pltpu


_TOKENS_PER_STEP = 8192
_ROWS_PER_ITER = 32


def _round_up(x, m):
    return (x + m - 1) // m * m


def _make_gather_body(tokens_per_step, d2):
    def _body(ids_ref, tbl_ref, out_ref):
        tok0 = pl.program_id(0) * tokens_per_step

        def chunk(c, carry):
            base = tok0 + c * _ROWS_PER_ITER
            row0 = c * _ROWS_PER_ITER
            rows = []
            for u in range(_ROWS_PER_ITER):
                rows.append(tbl_ref[ids_ref[base + u]][:, :d2])
            for g in range(_ROWS_PER_ITER // 8):
                out_ref[pl.ds(pl.multiple_of(row0 + 8 * g, 8), 8), :] = (
                    jnp.concatenate(rows[8 * g:8 * g + 8], axis=0))
            return carry

        jax.lax.fori_loop(0, tokens_per_step // _ROWS_PER_ITER, chunk, 0)

    return _body


@jax.jit
def kernel(embedding, feature_embed, item_feature, inputs, A, combined):
    del feature_embed, item_feature, A
    n_node = embedding.shape[0]
    n_pad, d2 = combined.shape
    b, s = inputs.shape
    n = b * s
    h2 = 2 * d2

    ids = jnp.clip(inputs.reshape(-1).astype(jnp.int32), 0, n_node - 1)
    np_tok = _round_up(n, _TOKENS_PER_STEP)
    if np_tok != n:
        ids = jnp.pad(ids, (0, np_tok - n))
    grid = (np_tok // _TOKENS_PER_STEP,)

    tbl = jnp.concatenate([combined, combined], axis=1).reshape(n_pad, 1, h2)

    out = pl.pallas_call(
        _make_gather_body(_TOKENS_PER_STEP, d2),
        out_shape=jax.ShapeDtypeStruct((np_tok, d2), jnp.float32),
        grid=grid,
        in_specs=[
            pl.BlockSpec(memory_space=pltpu.SMEM),
            pl.BlockSpec((n_pad, 1, h2), lambda i: (0, 0, 0)),
        ],
        out_specs=pl.BlockSpec((_TOKENS_PER_STEP, d2), lambda i: (i, 0)),
        compiler_params=pltpu.CompilerParams(
            dimension_semantics=("parallel",),
            vmem_limit_bytes=int(56 << 20),
        ),
    )(ids, tbl)

    return out[:n].reshape(b, s, d2)


# speedup vs baseline: 3.0555x; 1.1451x over previous
import jax
import jax.numpy as jnp
from jax.experimental import pallas as pl
from jax.experimental.pallas import tpu as pltpu


_TOKENS_PER_STEP = 8192
_ROWS_PER_ITER = 32


def _round_up(x, m):
    return (x + m - 1) // m * m


def _make_gather_body(tokens_per_step, d2):
    def _body(ids_ref, tbl_ref, out_ref):
        tok0 = pl.program_id(0) * tokens_per_step

        def chunk(c, carry):
            base = tok0 + c * _ROWS_PER_ITER
            row0 = c * _ROWS_PER_ITER
            rows = []
            for u in range(_ROWS_PER_ITER):
                rows.append(tbl_ref[ids_ref[base + u]])
            for g in range(_ROWS_PER_ITER // 8):
                out_ref[pl.ds(pl.multiple_of(row0 + 8 * g, 8), 8), :] = (
                    jnp.concatenate(rows[8 * g:8 * g + 8], axis=0))
            return carry

        jax.lax.fori_loop(0, tokens_per_step // _ROWS_PER_ITER, chunk, 0)

    return _body


@jax.jit
def kernel(embedding, feature_embed, item_feature, inputs, A, combined):
    del feature_embed, item_feature, A
    n_node = embedding.shape[0]
    n_pad, d2 = combined.shape
    b, s = inputs.shape
    n = b * s
    h2 = 2 * d2

    ids = jnp.clip(inputs.reshape(-1).astype(jnp.int32), 0, n_node - 1)
    np_tok = _round_up(n, _TOKENS_PER_STEP)
    if np_tok != n:
        ids = jnp.pad(ids, (0, np_tok - n))
    grid = (np_tok // _TOKENS_PER_STEP,)

    tbl = combined.reshape(n_pad, 1, d2)

    out = pl.pallas_call(
        _make_gather_body(_TOKENS_PER_STEP, d2),
        out_shape=jax.ShapeDtypeStruct((np_tok, d2), jnp.float32),
        grid=grid,
        in_specs=[
            pl.BlockSpec(memory_space=pltpu.SMEM),
            pl.BlockSpec((n_pad, 1, d2), lambda i: (0, 0, 0)),
        ],
        out_specs=pl.BlockSpec((_TOKENS_PER_STEP, d2), lambda i: (i, 0)),
        compiler_params=pltpu.CompilerParams(
            dimension_semantics=("parallel",),
            vmem_limit_bytes=int(56 << 20),
        ),
    )(ids, tbl)

    return out[:n].reshape(b, s, d2)
